# Initial kernel scaffold; baseline (speedup 1.0000x reference)
import jax
import jax.numpy as jnp
from jax.experimental import pallas as pl


def kernel(xyz, params):
    raise NotImplementedError("write your pallas kernel here")


# speedup vs baseline: 10.5020x; 10.5020x over previous
import functools

import jax
import jax.numpy as jnp
from jax.experimental import pallas as pl

_HI = jax.lax.Precision.HIGHEST
_EPS = 1e-5


def _bfdot(a, b):
    return jnp.dot(a.astype(jnp.bfloat16), b.astype(jnp.bfloat16),
                   preferred_element_type=jnp.float32)


def _fps_kernel(xs_ref, ys_ref, zs_ref, ox_ref, oy_ref, oz_ref, *, npoint):
    xs = xs_ref[...]
    ys = ys_ref[...]
    zs = zs_ref[...]
    B, N = xs.shape
    iota = jax.lax.broadcasted_iota(jnp.int32, (B, N), 1)

    def body(i, carry):
        dist_min, far = carry
        sel = iota == far
        cx = jnp.sum(jnp.where(sel, xs, 0.0), axis=1, keepdims=True)
        cy = jnp.sum(jnp.where(sel, ys, 0.0), axis=1, keepdims=True)
        cz = jnp.sum(jnp.where(sel, zs, 0.0), axis=1, keepdims=True)
        ox_ref[pl.ds(i, 1), :] = cx.reshape(1, B)
        oy_ref[pl.ds(i, 1), :] = cy.reshape(1, B)
        oz_ref[pl.ds(i, 1), :] = cz.reshape(1, B)
        d = (xs - cx) ** 2 + (ys - cy) ** 2 + (zs - cz) ** 2
        dist_min = jnp.minimum(dist_min, d)
        far = jnp.argmax(dist_min, axis=1).astype(jnp.int32)[:, None]
        return dist_min, far

    init = (jnp.full((B, N), 1e10, jnp.float32), jnp.zeros((B, 1), jnp.int32))
    jax.lax.fori_loop(0, npoint, body, init)


def _fps(xs, ys, zs, npoint):
    B, N = xs.shape
    shp = jax.ShapeDtypeStruct((npoint, B), jnp.float32)
    return pl.pallas_call(
        functools.partial(_fps_kernel, npoint=npoint),
        out_shape=[shp, shp, shp],
    )(xs, ys, zs)


def _cumsum_lanes(x):
    ST, N = x.shape
    s = 1
    while s < N:
        shifted = jnp.concatenate(
            [jnp.zeros((ST, s), x.dtype), x[:, :N - s]], axis=1)
        x = x + shifted
        s *= 2
    return x


def _group_kernel(nxyz_ref, xyzt_ref, vals_ref, w_ref, b_ref, y_ref, st_ref,
                  *, r2, K, ST):
    src = nxyz_ref[0]
    xyzt = xyzt_ref[0]
    vals = vals_ref[0]
    N = xyzt.shape[1]
    C = vals.shape[1]
    sx, sy, sz = src[:, 0:1], src[:, 1:2], src[:, 2:3]
    dx, dy, dz = xyzt[0:1, :], xyzt[1:2, :], xyzt[2:3, :]
    src2 = (sx * sx + sy * sy) + sz * sz
    dst2 = (dx * dx + dy * dy) + dz * dz

    def bf(v):
        return v.astype(jnp.bfloat16).astype(jnp.float32)

    cross = (bf(sx) * bf(dx) + bf(sy) * bf(dy)) + bf(sz) * bf(dz)
    d = (src2 + dst2) - 2.0 * cross
    inball = d <= r2
    ranks = _cumsum_lanes(inball.astype(jnp.int32))
    cnt = ranks[:, N - 1:N]
    kio = jax.lax.broadcasted_iota(jnp.int32, (1, K, 1), 1)
    sel = (ranks[:, None, :] == (kio + 1)) & inball[:, None, :]
    oh = sel.astype(jnp.float32).reshape(ST * K, N)
    g = jnp.dot(oh, vals, precision=_HI).reshape(ST, K, C)
    valid = jax.lax.broadcasted_iota(jnp.int32, (ST, K, 1), 1) < cnt[:, None, :]
    last = vals[N - 1:N, :][None, :, :]
    first = jnp.where(cnt[:, None, :] > 0, g[:, 0:1, :], last)
    g = jnp.where(valid, g, first)
    gx = g[:, :, :3] - src[:, None, :]
    x1 = jnp.concatenate([gx, g[:, :, 3:]], axis=-1).reshape(ST * K, C)
    y = _bfdot(x1, w_ref[...].T) + b_ref[...]

    @pl.when(jnp.logical_and(pl.program_id(0) == 0, pl.program_id(1) == 0))
    def _():
        st_ref[...] = jnp.zeros_like(st_ref)

    st_ref[0:1, :] += jnp.sum(y, axis=0, keepdims=True)
    st_ref[1:2, :] += jnp.sum(y * y, axis=0, keepdims=True)
    y_ref[0] = y


def _group_mlp1(new_xyz, xyzt, vals, W, b, r2, K, ST):
    B, S, _ = new_xyz.shape
    N, C = vals.shape[1], vals.shape[2]
    Cout = W.shape[0]
    y, stats = pl.pallas_call(
        functools.partial(_group_kernel, r2=r2, K=K, ST=ST),
        grid=(B, S // ST),
        in_specs=[
            pl.BlockSpec((1, ST, 3), lambda bb, ss: (bb, ss, 0)),
            pl.BlockSpec((1, 3, N), lambda bb, ss: (bb, 0, 0)),
            pl.BlockSpec((1, N, C), lambda bb, ss: (bb, 0, 0)),
            pl.BlockSpec((Cout, C), lambda bb, ss: (0, 0)),
            pl.BlockSpec((1, Cout), lambda bb, ss: (0, 0)),
        ],
        out_specs=[
            pl.BlockSpec((1, ST * K, Cout), lambda bb, ss: (bb, ss, 0)),
            pl.BlockSpec((8, Cout), lambda bb, ss: (0, 0)),
        ],
        out_shape=[
            jax.ShapeDtypeStruct((B, S * K, Cout), jnp.float32),
            jax.ShapeDtypeStruct((8, Cout), jnp.float32),
        ],
    )(new_xyz, xyzt, vals, W, b.reshape(1, -1))
    return y.reshape(B * S * K, Cout), stats


def _mlp_kernel(x_ref, a_ref, c_ref, w_ref, b_ref, y_ref, st_ref):
    h = jnp.maximum(x_ref[...] * a_ref[...] + c_ref[...], 0.0)
    y = _bfdot(h, w_ref[...].T) + b_ref[...]

    @pl.when(pl.program_id(0) == 0)
    def _():
        st_ref[...] = jnp.zeros_like(st_ref)

    st_ref[0:1, :] += jnp.sum(y, axis=0, keepdims=True)
    st_ref[1:2, :] += jnp.sum(y * y, axis=0, keepdims=True)
    y_ref[...] = y


def _mlp_layer(x, a, c, W, b, RT):
    R, Cin = x.shape
    Cout = W.shape[0]
    return pl.pallas_call(
        _mlp_kernel,
        grid=(R // RT,),
        in_specs=[
            pl.BlockSpec((RT, Cin), lambda i: (i, 0)),
            pl.BlockSpec((1, Cin), lambda i: (0, 0)),
            pl.BlockSpec((1, Cin), lambda i: (0, 0)),
            pl.BlockSpec((Cout, Cin), lambda i: (0, 0)),
            pl.BlockSpec((1, Cout), lambda i: (0, 0)),
        ],
        out_specs=[
            pl.BlockSpec((RT, Cout), lambda i: (i, 0)),
            pl.BlockSpec((8, Cout), lambda i: (0, 0)),
        ],
        out_shape=[
            jax.ShapeDtypeStruct((R, Cout), jnp.float32),
            jax.ShapeDtypeStruct((8, Cout), jnp.float32),
        ],
    )(x, a, c, W, b.reshape(1, -1))


def _pool_kernel(x_ref, a_ref, c_ref, o_ref):
    h = jnp.maximum(x_ref[...] * a_ref[...][None, :, :] + c_ref[...][None, :, :],
                    0.0)
    o_ref[...] = jnp.max(h, axis=1)


def _pool(x, a, c, T):
    BS, K, C = x.shape
    return pl.pallas_call(
        _pool_kernel,
        grid=(BS // T,),
        in_specs=[
            pl.BlockSpec((T, K, C), lambda i: (i, 0, 0)),
            pl.BlockSpec((1, C), lambda i: (0, 0)),
            pl.BlockSpec((1, C), lambda i: (0, 0)),
        ],
        out_specs=pl.BlockSpec((T, C), lambda i: (i, 0)),
        out_shape=jax.ShapeDtypeStruct((BS, C), jnp.float32),
    )(x, a, c)


def _bn_coeffs(stats, R, g, be):
    mean = stats[0] / R
    var = stats[1] / R - mean * mean
    a = g / jnp.sqrt(var + _EPS)
    c = be - mean * a
    return a.reshape(1, -1), c.reshape(1, -1)


def _bn_relu(y, g, be):
    mean = jnp.mean(y, axis=0, keepdims=True)
    var = jnp.mean((y - mean) ** 2, axis=0, keepdims=True)
    return jnp.maximum(g * (y - mean) / jnp.sqrt(var + _EPS) + be, 0.0)


def _tail_kernel(x_ref,
                 w1_ref, b1_ref, g1_ref, e1_ref,
                 w2_ref, b2_ref, g2_ref, e2_ref,
                 w3_ref, b3_ref, g3_ref, e3_ref,
                 fw1_ref, fb1_ref, bg_ref, bbe_ref,
                 fw2_ref, fb2_ref,
                 out_ref, l3_ref, *, B, S):
    x = x_ref[...]
    x = _bn_relu(_bfdot(x, w1_ref[...].T) + b1_ref[...],
                 g1_ref[...], e1_ref[...])
    x = _bn_relu(_bfdot(x, w2_ref[...].T) + b2_ref[...],
                 g2_ref[...], e2_ref[...])
    x = _bn_relu(_bfdot(x, w3_ref[...].T) + b3_ref[...],
                 g3_ref[...], e3_ref[...])
    C = x.shape[1]
    l3 = jnp.max(x.reshape(B, S, C), axis=1)
    l3_ref[...] = l3
    h = _bfdot(l3, fw1_ref[...].T) + fb1_ref[...]
    mean = jnp.mean(h, axis=0, keepdims=True)
    var = jnp.mean((h - mean) ** 2, axis=0, keepdims=True)
    h = jnp.maximum(bg_ref[...] * (h - mean) / jnp.sqrt(var + _EPS)
                    + bbe_ref[...], 0.0)
    lg = _bfdot(h, fw2_ref[...].T) + fb2_ref[...]
    m = jnp.max(lg, axis=-1, keepdims=True)
    s = lg - m
    out_ref[...] = s - jnp.log(jnp.sum(jnp.exp(s), axis=-1, keepdims=True))


def _sa_stage(new_xyz, xyzt, vals, layer_params, r2, K, ST, RT):
    (W1, b1, g1, be1), (W2, b2, g2, be2), (W3, b3, g3, be3) = layer_params
    y1, s1 = _group_mlp1(new_xyz, xyzt, vals, W1, b1, r2, K, ST)
    R = y1.shape[0]
    a1, c1 = _bn_coeffs(s1, R, g1, be1)
    y2, s2 = _mlp_layer(y1, a1, c1, W2, b2, RT)
    a2, c2 = _bn_coeffs(s2, R, g2, be2)
    y3, s3 = _mlp_layer(y2, a2, c2, W3, b3, RT)
    a3, c3 = _bn_coeffs(s3, R, g3, be3)
    C3 = W3.shape[0]
    pooled = _pool(y3.reshape(R // K, K, C3), a3, c3, max(8, (R // K) // 32))
    return pooled


def kernel(xyz, params):
    B = xyz.shape[0]
    N = xyz.shape[2]
    xs, ys, zs = xyz[:, 0, :], xyz[:, 1, :], xyz[:, 2, :]

    ox1, oy1, oz1 = _fps(xs, ys, zs, 512)
    new_xyz1 = jnp.stack([ox1.T, oy1.T, oz1.T], axis=-1)
    xyzt1 = xyz[:, 0:3, :]
    vals1 = jnp.transpose(xyz, (0, 2, 1))
    l1_points = _sa_stage(new_xyz1, xyzt1, vals1, params['sa1'],
                          0.2 ** 2, 32, 32, 4096)
    l1_points = l1_points.reshape(B, 512, 64)

    xs2, ys2, zs2 = ox1.T, oy1.T, oz1.T
    ox2, oy2, oz2 = _fps(xs2, ys2, zs2, 128)
    new_xyz2 = jnp.stack([ox2.T, oy2.T, oz2.T], axis=-1)
    xyzt2 = jnp.stack([xs2, ys2, zs2], axis=1)
    vals2 = jnp.concatenate([new_xyz1, l1_points], axis=-1)
    l2_points = _sa_stage(new_xyz2, xyzt2, vals2, params['sa2'],
                          0.4 ** 2, 64, 32, 4096)
    l2_points = l2_points.reshape(B, 128, 128)

    x3 = jnp.concatenate([new_xyz2, l2_points], axis=-1).reshape(B * 128, 131)
    (W1, b1, g1, be1), (W2, b2, g2, be2), (W3, b3, g3, be3) = params['sa3']
    r2 = lambda v: v.reshape(1, -1)
    logits, l3 = pl.pallas_call(
        functools.partial(_tail_kernel, B=B, S=128),
        out_shape=[
            jax.ShapeDtypeStruct((B, 40), jnp.float32),
            jax.ShapeDtypeStruct((B, 512), jnp.float32),
        ],
    )(x3,
      W1, r2(b1), r2(g1), r2(be1),
      W2, r2(b2), r2(g2), r2(be2),
      W3, r2(b3), r2(g3), r2(be3),
      params['fc1_w'], r2(params['fc1_b']),
      r2(params['bn1_g']), r2(params['bn1_be']),
      params['fc2_w'], r2(params['fc2_b']))
    return (logits, l3[:, :, None])


# speedup vs baseline: 18.3367x; 1.7460x over previous
import functools

import jax
import jax.numpy as jnp
from jax.experimental import pallas as pl

_HI = jax.lax.Precision.HIGHEST
_EPS = 1e-5


def _bfdot(a, b):
    return jnp.dot(a.astype(jnp.bfloat16), b.astype(jnp.bfloat16),
                   preferred_element_type=jnp.float32)


def _fps_kernel(xs_ref, ys_ref, zs_ref, ox_ref, oy_ref, oz_ref, *, npoint):
    xs = xs_ref[...]
    ys = ys_ref[...]
    zs = zs_ref[...]
    B, N = xs.shape
    iota = jax.lax.broadcasted_iota(jnp.int32, (B, N), 1)

    def body(i, carry):
        dist_min, far = carry
        sel = iota == far
        cx = jnp.sum(jnp.where(sel, xs, 0.0), axis=1, keepdims=True)
        cy = jnp.sum(jnp.where(sel, ys, 0.0), axis=1, keepdims=True)
        cz = jnp.sum(jnp.where(sel, zs, 0.0), axis=1, keepdims=True)
        ox_ref[pl.ds(i, 1), :] = cx.reshape(1, B)
        oy_ref[pl.ds(i, 1), :] = cy.reshape(1, B)
        oz_ref[pl.ds(i, 1), :] = cz.reshape(1, B)
        d = (xs - cx) ** 2 + (ys - cy) ** 2 + (zs - cz) ** 2
        dist_min = jnp.minimum(dist_min, d)
        far = jnp.argmax(dist_min, axis=1).astype(jnp.int32)[:, None]
        return dist_min, far

    init = (jnp.full((B, N), 1e10, jnp.float32), jnp.zeros((B, 1), jnp.int32))
    jax.lax.fori_loop(0, npoint, body, init)


def _fps(xs, ys, zs, npoint):
    B, N = xs.shape
    shp = jax.ShapeDtypeStruct((npoint, B), jnp.float32)
    return pl.pallas_call(
        functools.partial(_fps_kernel, npoint=npoint),
        out_shape=[shp, shp, shp],
    )(xs, ys, zs)


def _cumsum_lanes(x):
    ST, N = x.shape
    s = 1
    while s < N:
        shifted = jnp.concatenate(
            [jnp.zeros((ST, s), x.dtype), x[:, :N - s]], axis=1)
        x = x + shifted
        s *= 2
    return x


def _group_kernel(nxyz_ref, xyzt_ref, vals_ref, w_ref, b_ref, y_ref, st_ref,
                  *, r2, K, ST):
    src = nxyz_ref[0]
    xyzt = xyzt_ref[0]
    vals = vals_ref[0]
    N = xyzt.shape[1]
    Ccat = vals.shape[1]
    sx, sy, sz = src[:, 0:1], src[:, 1:2], src[:, 2:3]
    dx, dy, dz = xyzt[0:1, :], xyzt[1:2, :], xyzt[2:3, :]
    src2 = (sx * sx + sy * sy) + sz * sz
    dst2 = (dx * dx + dy * dy) + dz * dz

    def bf(v):
        return v.astype(jnp.bfloat16).astype(jnp.float32)

    cross = (bf(sx) * bf(dx) + bf(sy) * bf(dy)) + bf(sz) * bf(dz)
    d = (src2 + dst2) - 2.0 * cross
    inball = d <= r2
    ranks = _cumsum_lanes(inball.astype(jnp.int32))
    cnt = ranks[:, N - 1:N]
    kio = jax.lax.broadcasted_iota(jnp.int32, (1, K, 1), 1)
    sel = (ranks[:, None, :] == (kio + 1)) & inball[:, None, :]
    oh = sel.astype(jnp.bfloat16).reshape(ST * K, N)
    g = jnp.dot(oh, vals.astype(jnp.bfloat16),
                preferred_element_type=jnp.float32).reshape(ST, K, Ccat)
    valid = jax.lax.broadcasted_iota(jnp.int32, (ST, K, 1), 1) < cnt[:, None, :]
    last = vals[N - 1:N, :][None, :, :]
    first = jnp.where(cnt[:, None, :] > 0, g[:, 0:1, :], last)
    g = jnp.where(valid, g, first)
    gx = (g[:, :, 0:3] + g[:, :, 3:6]) + g[:, :, 6:9]
    gx = gx - src[:, None, :]
    x1 = jnp.concatenate([gx, g[:, :, 9:]], axis=-1).reshape(ST * K, Ccat - 6)
    y = _bfdot(x1, w_ref[...].T) + b_ref[...]

    @pl.when(jnp.logical_and(pl.program_id(0) == 0, pl.program_id(1) == 0))
    def _():
        st_ref[...] = jnp.zeros_like(st_ref)

    st_ref[0:1, :] += jnp.sum(y, axis=0, keepdims=True)
    st_ref[1:2, :] += jnp.sum(y * y, axis=0, keepdims=True)
    y_ref[0] = y


def _group_mlp1(new_xyz, xyzt, vals, W, b, r2, K, ST):
    B, S, _ = new_xyz.shape
    N, C = vals.shape[1], vals.shape[2]
    Cout, Cin = W.shape
    y, stats = pl.pallas_call(
        functools.partial(_group_kernel, r2=r2, K=K, ST=ST),
        grid=(B, S // ST),
        in_specs=[
            pl.BlockSpec((1, ST, 3), lambda bb, ss: (bb, ss, 0)),
            pl.BlockSpec((1, 3, N), lambda bb, ss: (bb, 0, 0)),
            pl.BlockSpec((1, N, C), lambda bb, ss: (bb, 0, 0)),
            pl.BlockSpec((Cout, Cin), lambda bb, ss: (0, 0)),
            pl.BlockSpec((1, Cout), lambda bb, ss: (0, 0)),
        ],
        out_specs=[
            pl.BlockSpec((1, ST * K, Cout), lambda bb, ss: (bb, ss, 0)),
            pl.BlockSpec((8, Cout), lambda bb, ss: (0, 0)),
        ],
        out_shape=[
            jax.ShapeDtypeStruct((B, S * K, Cout), jnp.float32),
            jax.ShapeDtypeStruct((8, Cout), jnp.float32),
        ],
    )(new_xyz, xyzt, vals, W, b.reshape(1, -1))
    return y.reshape(B * S * K, Cout), stats


def _mlp_kernel(x_ref, a_ref, c_ref, w_ref, b_ref, y_ref, st_ref):
    h = jnp.maximum(x_ref[...] * a_ref[...] + c_ref[...], 0.0)
    y = _bfdot(h, w_ref[...].T) + b_ref[...]

    @pl.when(pl.program_id(0) == 0)
    def _():
        st_ref[...] = jnp.zeros_like(st_ref)

    st_ref[0:1, :] += jnp.sum(y, axis=0, keepdims=True)
    st_ref[1:2, :] += jnp.sum(y * y, axis=0, keepdims=True)
    y_ref[...] = y


def _mlp_layer(x, a, c, W, b, RT):
    R, Cin = x.shape
    Cout = W.shape[0]
    return pl.pallas_call(
        _mlp_kernel,
        grid=(R // RT,),
        in_specs=[
            pl.BlockSpec((RT, Cin), lambda i: (i, 0)),
            pl.BlockSpec((1, Cin), lambda i: (0, 0)),
            pl.BlockSpec((1, Cin), lambda i: (0, 0)),
            pl.BlockSpec((Cout, Cin), lambda i: (0, 0)),
            pl.BlockSpec((1, Cout), lambda i: (0, 0)),
        ],
        out_specs=[
            pl.BlockSpec((RT, Cout), lambda i: (i, 0)),
            pl.BlockSpec((8, Cout), lambda i: (0, 0)),
        ],
        out_shape=[
            jax.ShapeDtypeStruct((R, Cout), jnp.float32),
            jax.ShapeDtypeStruct((8, Cout), jnp.float32),
        ],
    )(x, a, c, W, b.reshape(1, -1))


def _pool_kernel(x_ref, a_ref, c_ref, o_ref):
    h = jnp.maximum(x_ref[...] * a_ref[...][None, :, :] + c_ref[...][None, :, :],
                    0.0)
    o_ref[...] = jnp.max(h, axis=1)


def _pool(x, a, c, T):
    BS, K, C = x.shape
    return pl.pallas_call(
        _pool_kernel,
        grid=(BS // T,),
        in_specs=[
            pl.BlockSpec((T, K, C), lambda i: (i, 0, 0)),
            pl.BlockSpec((1, C), lambda i: (0, 0)),
            pl.BlockSpec((1, C), lambda i: (0, 0)),
        ],
        out_specs=pl.BlockSpec((T, C), lambda i: (i, 0)),
        out_shape=jax.ShapeDtypeStruct((BS, C), jnp.float32),
    )(x, a, c)


def _split_xyz_vals(xyzv, pts):
    hi = xyzv.astype(jnp.bfloat16).astype(jnp.float32)
    r = xyzv - hi
    mid = r.astype(jnp.bfloat16).astype(jnp.float32)
    lo = r - mid
    return jnp.concatenate([hi, mid, lo, pts], axis=-1)


def _bn_coeffs(stats, R, g, be):
    mean = stats[0] / R
    var = stats[1] / R - mean * mean
    a = g / jnp.sqrt(var + _EPS)
    c = be - mean * a
    return a.reshape(1, -1), c.reshape(1, -1)


def _bn_relu(y, g, be):
    mean = jnp.mean(y, axis=0, keepdims=True)
    var = jnp.mean((y - mean) ** 2, axis=0, keepdims=True)
    return jnp.maximum(g * (y - mean) / jnp.sqrt(var + _EPS) + be, 0.0)


def _tail_kernel(x_ref,
                 w1_ref, b1_ref, g1_ref, e1_ref,
                 w2_ref, b2_ref, g2_ref, e2_ref,
                 w3_ref, b3_ref, g3_ref, e3_ref,
                 fw1_ref, fb1_ref, bg_ref, bbe_ref,
                 fw2_ref, fb2_ref,
                 out_ref, l3_ref, *, B, S):
    x = x_ref[...]
    x = _bn_relu(_bfdot(x, w1_ref[...].T) + b1_ref[...],
                 g1_ref[...], e1_ref[...])
    x = _bn_relu(_bfdot(x, w2_ref[...].T) + b2_ref[...],
                 g2_ref[...], e2_ref[...])
    x = _bn_relu(_bfdot(x, w3_ref[...].T) + b3_ref[...],
                 g3_ref[...], e3_ref[...])
    C = x.shape[1]
    l3 = jnp.max(x.reshape(B, S, C), axis=1)
    l3_ref[...] = l3
    h = _bfdot(l3, fw1_ref[...].T) + fb1_ref[...]
    mean = jnp.mean(h, axis=0, keepdims=True)
    var = jnp.mean((h - mean) ** 2, axis=0, keepdims=True)
    h = jnp.maximum(bg_ref[...] * (h - mean) / jnp.sqrt(var + _EPS)
                    + bbe_ref[...], 0.0)
    lg = _bfdot(h, fw2_ref[...].T) + fb2_ref[...]
    m = jnp.max(lg, axis=-1, keepdims=True)
    s = lg - m
    out_ref[...] = s - jnp.log(jnp.sum(jnp.exp(s), axis=-1, keepdims=True))


def _sa_stage(new_xyz, xyzt, vals, layer_params, r2, K, ST, RT):
    (W1, b1, g1, be1), (W2, b2, g2, be2), (W3, b3, g3, be3) = layer_params
    y1, s1 = _group_mlp1(new_xyz, xyzt, vals, W1, b1, r2, K, ST)
    R = y1.shape[0]
    a1, c1 = _bn_coeffs(s1, R, g1, be1)
    y2, s2 = _mlp_layer(y1, a1, c1, W2, b2, RT)
    a2, c2 = _bn_coeffs(s2, R, g2, be2)
    y3, s3 = _mlp_layer(y2, a2, c2, W3, b3, RT)
    a3, c3 = _bn_coeffs(s3, R, g3, be3)
    C3 = W3.shape[0]
    pooled = _pool(y3.reshape(R // K, K, C3), a3, c3, max(8, (R // K) // 32))
    return pooled


def kernel(xyz, params):
    B = xyz.shape[0]
    N = xyz.shape[2]
    xs, ys, zs = xyz[:, 0, :], xyz[:, 1, :], xyz[:, 2, :]

    ox1, oy1, oz1 = _fps(xs, ys, zs, 512)
    new_xyz1 = jnp.stack([ox1.T, oy1.T, oz1.T], axis=-1)
    xyzt1 = xyz[:, 0:3, :]
    valst = jnp.transpose(xyz, (0, 2, 1))
    vals1 = _split_xyz_vals(valst[:, :, :3], valst[:, :, 3:])
    l1_points = _sa_stage(new_xyz1, xyzt1, vals1, params['sa1'],
                          0.2 ** 2, 32, 32, 4096)
    l1_points = l1_points.reshape(B, 512, 64)

    xs2, ys2, zs2 = ox1.T, oy1.T, oz1.T
    ox2, oy2, oz2 = _fps(xs2, ys2, zs2, 128)
    new_xyz2 = jnp.stack([ox2.T, oy2.T, oz2.T], axis=-1)
    xyzt2 = jnp.stack([xs2, ys2, zs2], axis=1)
    vals2 = _split_xyz_vals(new_xyz1, l1_points)
    l2_points = _sa_stage(new_xyz2, xyzt2, vals2, params['sa2'],
                          0.4 ** 2, 64, 32, 4096)
    l2_points = l2_points.reshape(B, 128, 128)

    x3 = jnp.concatenate([new_xyz2, l2_points], axis=-1).reshape(B * 128, 131)
    (W1, b1, g1, be1), (W2, b2, g2, be2), (W3, b3, g3, be3) = params['sa3']
    r2 = lambda v: v.reshape(1, -1)
    logits, l3 = pl.pallas_call(
        functools.partial(_tail_kernel, B=B, S=128),
        out_shape=[
            jax.ShapeDtypeStruct((B, 40), jnp.float32),
            jax.ShapeDtypeStruct((B, 512), jnp.float32),
        ],
    )(x3,
      W1, r2(b1), r2(g1), r2(be1),
      W2, r2(b2), r2(g2), r2(be2),
      W3, r2(b3), r2(g3), r2(be3),
      params['fc1_w'], r2(params['fc1_b']),
      r2(params['bn1_g']), r2(params['bn1_be']),
      params['fc2_w'], r2(params['fc2_b']))
    return (logits, l3[:, :, None])


# speedup vs baseline: 18.8212x; 1.0264x over previous
import functools

import jax
import jax.numpy as jnp
from jax.experimental import pallas as pl

_HI = jax.lax.Precision.HIGHEST
_EPS = 1e-5


def _bfdot(a, b):
    return jnp.dot(a.astype(jnp.bfloat16), b.astype(jnp.bfloat16),
                   preferred_element_type=jnp.float32)


def _fps_kernel(xs_ref, ys_ref, zs_ref, ox_ref, oy_ref, oz_ref, *, npoint):
    xs = xs_ref[...]
    ys = ys_ref[...]
    zs = zs_ref[...]
    B, N = xs.shape
    iota = jax.lax.broadcasted_iota(jnp.int32, (B, N), 1)

    def body(i, carry):
        dist_min, far = carry
        sel = iota == far
        cx = jnp.sum(jnp.where(sel, xs, 0.0), axis=1, keepdims=True)
        cy = jnp.sum(jnp.where(sel, ys, 0.0), axis=1, keepdims=True)
        cz = jnp.sum(jnp.where(sel, zs, 0.0), axis=1, keepdims=True)
        ox_ref[pl.ds(i, 1), :] = cx.reshape(1, B)
        oy_ref[pl.ds(i, 1), :] = cy.reshape(1, B)
        oz_ref[pl.ds(i, 1), :] = cz.reshape(1, B)
        d = (xs - cx) ** 2 + (ys - cy) ** 2 + (zs - cz) ** 2
        dist_min = jnp.minimum(dist_min, d)
        far = jnp.argmax(dist_min, axis=1).astype(jnp.int32)[:, None]
        return dist_min, far

    init = (jnp.full((B, N), 1e10, jnp.float32), jnp.zeros((B, 1), jnp.int32))
    jax.lax.fori_loop(0, npoint, body, init)


def _fps(xs, ys, zs, npoint):
    B, N = xs.shape
    shp = jax.ShapeDtypeStruct((npoint, B), jnp.float32)
    return pl.pallas_call(
        functools.partial(_fps_kernel, npoint=npoint),
        out_shape=[shp, shp, shp],
    )(xs, ys, zs)


def _cumsum_lanes(x):
    ST, N = x.shape
    s = 1
    while s < N:
        shifted = jnp.concatenate(
            [jnp.zeros((ST, s), x.dtype), x[:, :N - s]], axis=1)
        x = x + shifted
        s *= 2
    return x


def _group_kernel(nxyz_ref, xyzt_ref, vals_ref, w_ref, b_ref, y_ref, st_ref,
                  *, r2, K, ST):
    src = nxyz_ref[0]
    xyzt = xyzt_ref[0]
    vals = vals_ref[0]
    N = xyzt.shape[1]
    C = vals.shape[1]
    sx, sy, sz = src[:, 0:1], src[:, 1:2], src[:, 2:3]
    dx, dy, dz = xyzt[0:1, :], xyzt[1:2, :], xyzt[2:3, :]
    src2 = (sx * sx + sy * sy) + sz * sz
    dst2 = (dx * dx + dy * dy) + dz * dz

    def bf(v):
        return v.astype(jnp.bfloat16).astype(jnp.float32)

    cross = (bf(sx) * bf(dx) + bf(sy) * bf(dy)) + bf(sz) * bf(dz)
    d = (src2 + dst2) - 2.0 * cross
    inball = d <= r2
    ranks = _cumsum_lanes(inball.astype(jnp.int32))
    cnt = ranks[:, N - 1:N]
    jio = jax.lax.broadcasted_iota(jnp.int32, (1, N), 1)
    rm = jnp.where(inball, ranks, 0)
    rm = jnp.where((cnt == 0) & (jio == N - 1), 1, rm)
    kio = jax.lax.broadcasted_iota(jnp.int32, (1, K, 1), 1)
    sel = rm[:, None, :] == (kio + 1)
    oh = sel.astype(jnp.bfloat16).reshape(ST * K, N)
    vh = vals.astype(jnp.bfloat16)
    r1 = vals - vh.astype(jnp.float32)
    vm = r1.astype(jnp.bfloat16)
    vl = (r1 - vm.astype(jnp.float32)).astype(jnp.bfloat16)
    vcat = jnp.concatenate([vh, vm, vl], axis=-1)
    g = jnp.dot(oh, vcat,
                preferred_element_type=jnp.float32).reshape(ST, K, 3 * C)
    valid = jax.lax.broadcasted_iota(jnp.int32, (ST, K, 1), 1) < cnt[:, None, :]
    g = jnp.where(valid, g, g[:, 0:1, :])
    ge = (g[:, :, 0:C] + g[:, :, C:2 * C]) + g[:, :, 2 * C:]
    gx = ge[:, :, 0:3] - src[:, None, :]
    x1 = jnp.concatenate([gx, ge[:, :, 3:]], axis=-1).reshape(ST * K, C)
    y = _bfdot(x1, w_ref[...].T) + b_ref[...]

    @pl.when(jnp.logical_and(pl.program_id(0) == 0, pl.program_id(1) == 0))
    def _():
        st_ref[...] = jnp.zeros_like(st_ref)

    st_ref[0:1, :] += jnp.sum(y, axis=0, keepdims=True)
    st_ref[1:2, :] += jnp.sum(y * y, axis=0, keepdims=True)
    y_ref[0] = y


def _group_mlp1(new_xyz, xyzt, vals, W, b, r2, K, ST):
    B, S, _ = new_xyz.shape
    N, C = vals.shape[1], vals.shape[2]
    Cout, Cin = W.shape
    y, stats = pl.pallas_call(
        functools.partial(_group_kernel, r2=r2, K=K, ST=ST),
        grid=(B, S // ST),
        in_specs=[
            pl.BlockSpec((1, ST, 3), lambda bb, ss: (bb, ss, 0)),
            pl.BlockSpec((1, 3, N), lambda bb, ss: (bb, 0, 0)),
            pl.BlockSpec((1, N, C), lambda bb, ss: (bb, 0, 0)),
            pl.BlockSpec((Cout, Cin), lambda bb, ss: (0, 0)),
            pl.BlockSpec((1, Cout), lambda bb, ss: (0, 0)),
        ],
        out_specs=[
            pl.BlockSpec((1, ST * K, Cout), lambda bb, ss: (bb, ss, 0)),
            pl.BlockSpec((8, Cout), lambda bb, ss: (0, 0)),
        ],
        out_shape=[
            jax.ShapeDtypeStruct((B, S * K, Cout), jnp.float32),
            jax.ShapeDtypeStruct((8, Cout), jnp.float32),
        ],
    )(new_xyz, xyzt, vals, W, b.reshape(1, -1))
    return y.reshape(B * S * K, Cout), stats


def _mlp_kernel(x_ref, sp_ref, g_ref, be_ref, w_ref, b_ref, y_ref, st_ref,
                *, R):
    mean = sp_ref[0:1, :] / R
    var = sp_ref[1:2, :] / R - mean * mean
    a = g_ref[...] / jnp.sqrt(var + _EPS)
    c = be_ref[...] - mean * a
    h = jnp.maximum(x_ref[...] * a + c, 0.0)
    y = _bfdot(h, w_ref[...].T) + b_ref[...]

    @pl.when(pl.program_id(0) == 0)
    def _():
        st_ref[...] = jnp.zeros_like(st_ref)

    st_ref[0:1, :] += jnp.sum(y, axis=0, keepdims=True)
    st_ref[1:2, :] += jnp.sum(y * y, axis=0, keepdims=True)
    y_ref[...] = y


def _mlp_layer(x, sp, g, be, W, b, RT):
    R, Cin = x.shape
    Cout = W.shape[0]
    return pl.pallas_call(
        functools.partial(_mlp_kernel, R=R),
        grid=(R // RT,),
        in_specs=[
            pl.BlockSpec((RT, Cin), lambda i: (i, 0)),
            pl.BlockSpec((8, Cin), lambda i: (0, 0)),
            pl.BlockSpec((1, Cin), lambda i: (0, 0)),
            pl.BlockSpec((1, Cin), lambda i: (0, 0)),
            pl.BlockSpec((Cout, Cin), lambda i: (0, 0)),
            pl.BlockSpec((1, Cout), lambda i: (0, 0)),
        ],
        out_specs=[
            pl.BlockSpec((RT, Cout), lambda i: (i, 0)),
            pl.BlockSpec((8, Cout), lambda i: (0, 0)),
        ],
        out_shape=[
            jax.ShapeDtypeStruct((R, Cout), jnp.float32),
            jax.ShapeDtypeStruct((8, Cout), jnp.float32),
        ],
    )(x, sp, g.reshape(1, -1), be.reshape(1, -1), W, b.reshape(1, -1))


def _pool_kernel(x_ref, sp_ref, g_ref, be_ref, o_ref, *, R):
    mean = sp_ref[0:1, :] / R
    var = sp_ref[1:2, :] / R - mean * mean
    a = (g_ref[...] / jnp.sqrt(var + _EPS))[None, :, :]
    c = (be_ref[...] - mean * (g_ref[...] / jnp.sqrt(var + _EPS)))[None, :, :]
    h = jnp.maximum(x_ref[...] * a + c, 0.0)
    o_ref[...] = jnp.max(h, axis=1)


def _pool(x, sp, g, be, T, R):
    BS, K, C = x.shape
    return pl.pallas_call(
        functools.partial(_pool_kernel, R=R),
        grid=(BS // T,),
        in_specs=[
            pl.BlockSpec((T, K, C), lambda i: (i, 0, 0)),
            pl.BlockSpec((8, C), lambda i: (0, 0)),
            pl.BlockSpec((1, C), lambda i: (0, 0)),
            pl.BlockSpec((1, C), lambda i: (0, 0)),
        ],
        out_specs=pl.BlockSpec((T, C), lambda i: (i, 0)),
        out_shape=jax.ShapeDtypeStruct((BS, C), jnp.float32),
    )(x, sp, g.reshape(1, -1), be.reshape(1, -1))


def _bn_relu(y, g, be):
    mean = jnp.mean(y, axis=0, keepdims=True)
    var = jnp.mean((y - mean) ** 2, axis=0, keepdims=True)
    return jnp.maximum(g * (y - mean) / jnp.sqrt(var + _EPS) + be, 0.0)


def _tail_kernel(x_ref,
                 w1_ref, b1_ref, g1_ref, e1_ref,
                 w2_ref, b2_ref, g2_ref, e2_ref,
                 w3_ref, b3_ref, g3_ref, e3_ref,
                 fw1_ref, fb1_ref, bg_ref, bbe_ref,
                 fw2_ref, fb2_ref,
                 out_ref, l3_ref, *, B, S):
    x = x_ref[...]
    x = _bn_relu(_bfdot(x, w1_ref[...].T) + b1_ref[...],
                 g1_ref[...], e1_ref[...])
    x = _bn_relu(_bfdot(x, w2_ref[...].T) + b2_ref[...],
                 g2_ref[...], e2_ref[...])
    x = _bn_relu(_bfdot(x, w3_ref[...].T) + b3_ref[...],
                 g3_ref[...], e3_ref[...])
    C = x.shape[1]
    l3 = jnp.max(x.reshape(B, S, C), axis=1)
    l3_ref[...] = l3
    h = _bfdot(l3, fw1_ref[...].T) + fb1_ref[...]
    mean = jnp.mean(h, axis=0, keepdims=True)
    var = jnp.mean((h - mean) ** 2, axis=0, keepdims=True)
    h = jnp.maximum(bg_ref[...] * (h - mean) / jnp.sqrt(var + _EPS)
                    + bbe_ref[...], 0.0)
    lg = _bfdot(h, fw2_ref[...].T) + fb2_ref[...]
    m = jnp.max(lg, axis=-1, keepdims=True)
    s = lg - m
    out_ref[...] = s - jnp.log(jnp.sum(jnp.exp(s), axis=-1, keepdims=True))


def _sa_stage(new_xyz, xyzt, vals, layer_params, r2, K, ST, RT):
    (W1, b1, g1, be1), (W2, b2, g2, be2), (W3, b3, g3, be3) = layer_params
    y1, s1 = _group_mlp1(new_xyz, xyzt, vals, W1, b1, r2, K, ST)
    R = y1.shape[0]
    y2, s2 = _mlp_layer(y1, s1, g1, be1, W2, b2, RT)
    y3, s3 = _mlp_layer(y2, s2, g2, be2, W3, b3, RT)
    C3 = W3.shape[0]
    pooled = _pool(y3.reshape(R // K, K, C3), s3, g3, be3,
                   max(8, (R // K) // 32), R)
    return pooled


def kernel(xyz, params):
    B = xyz.shape[0]
    N = xyz.shape[2]
    xs, ys, zs = xyz[:, 0, :], xyz[:, 1, :], xyz[:, 2, :]

    ox1, oy1, oz1 = _fps(xs, ys, zs, 512)
    new_xyz1 = jnp.stack([ox1.T, oy1.T, oz1.T], axis=-1)
    xyzt1 = xyz[:, 0:3, :]
    vals1 = jnp.transpose(xyz, (0, 2, 1))
    l1_points = _sa_stage(new_xyz1, xyzt1, vals1, params['sa1'],
                          0.2 ** 2, 32, 32, 4096)
    l1_points = l1_points.reshape(B, 512, 64)

    xs2, ys2, zs2 = ox1.T, oy1.T, oz1.T
    ox2, oy2, oz2 = _fps(xs2, ys2, zs2, 128)
    new_xyz2 = jnp.stack([ox2.T, oy2.T, oz2.T], axis=-1)
    xyzt2 = jnp.stack([xs2, ys2, zs2], axis=1)
    vals2 = jnp.concatenate([new_xyz1, l1_points], axis=-1)
    l2_points = _sa_stage(new_xyz2, xyzt2, vals2, params['sa2'],
                          0.4 ** 2, 64, 32, 4096)
    l2_points = l2_points.reshape(B, 128, 128)

    x3 = jnp.concatenate([new_xyz2, l2_points], axis=-1).reshape(B * 128, 131)
    (W1, b1, g1, be1), (W2, b2, g2, be2), (W3, b3, g3, be3) = params['sa3']
    r2 = lambda v: v.reshape(1, -1)
    logits, l3 = pl.pallas_call(
        functools.partial(_tail_kernel, B=B, S=128),
        out_shape=[
            jax.ShapeDtypeStruct((B, 40), jnp.float32),
            jax.ShapeDtypeStruct((B, 512), jnp.float32),
        ],
    )(x3,
      W1, r2(b1), r2(g1), r2(be1),
      W2, r2(b2), r2(g2), r2(be2),
      W3, r2(b3), r2(g3), r2(be3),
      params['fc1_w'], r2(params['fc1_b']),
      r2(params['bn1_g']), r2(params['bn1_be']),
      params['fc2_w'], r2(params['fc2_b']))
    return (logits, l3[:, :, None])
